# baseline (speedup 1.0000x reference)
import functools

import jax
import jax.numpy as jnp
from jax import lax
from jax.experimental import pallas as pl
from jax.experimental.pallas import tpu as pltpu
from jax.experimental.pallas import tpu_sc as plsc

B, T, C = 1, 2048, 1024
E, K, FFN = 16, 2, 1024
S = B * T
PAIRS = S * K
TM = 128
NT = PAIRS // TM + E
NSLOT = NT * TM

NC, NS = 2, 16
NW = NC * NS
RING = 4


def _router_body(x_ref, wr_ref, posk_ref, w0b_ref, w1b_ref, ti_ref, hdr_ref):
    x = x_ref[...]
    logits = jnp.dot(x, wr_ref[...], preferred_element_type=jnp.float32)

    eidx = lax.broadcasted_iota(jnp.int32, (S, E), 1)
    m1 = jnp.max(logits, axis=1, keepdims=True)
    a1 = jnp.min(jnp.where(logits == m1, eidx, E), axis=1, keepdims=True)
    masked = jnp.where(eidx == a1, -jnp.inf, logits)
    m2 = jnp.max(masked, axis=1, keepdims=True)
    a2 = jnp.min(jnp.where(masked == m2, eidx, E), axis=1, keepdims=True)

    e2 = jnp.exp(m2 - m1)
    w0 = 1.0 / (1.0 + e2)
    w1 = e2 / (1.0 + e2)

    oh1 = (eidx == a1).astype(jnp.float32)
    oh2 = (eidx == a2).astype(jnp.float32)
    oh = jnp.concatenate([oh1, oh2], axis=1)

    ch = 256
    ri = lax.broadcasted_iota(jnp.int32, (ch, ch), 0)
    ci = lax.broadcasted_iota(jnp.int32, (ch, ch), 1)
    tri = (ri > ci).astype(jnp.float32)
    carry = jnp.zeros((1, 2 * E), jnp.float32)
    parts = []
    for b in range(S // ch):
        blk = oh[b * ch:(b + 1) * ch, :]
        parts.append(jnp.dot(tri, blk, preferred_element_type=jnp.float32)
                     + carry)
        carry = carry + jnp.sum(blk, axis=0, keepdims=True)
    csum = jnp.concatenate(parts, axis=0)

    cnt = carry
    cnt1, cnt2 = cnt[:, :E], cnt[:, E:]
    cnt_tot = (cnt1 + cnt2).astype(jnp.int32)
    cap = ((cnt_tot + (TM - 1)) // TM) * TM

    eli = lax.broadcasted_iota(jnp.int32, (E, E), 0)
    eci = lax.broadcasted_iota(jnp.int32, (E, E), 1)
    lt = (eli < eci).astype(jnp.float32)
    offs = jnp.dot(cap.astype(jnp.float32), lt,
                   preferred_element_type=jnp.float32)

    rank1 = jnp.sum(csum[:, :E] * oh1, axis=1, keepdims=True)
    rank2 = jnp.sum((csum[:, E:] + cnt1) * oh2, axis=1, keepdims=True)
    off1 = jnp.sum(offs * oh1, axis=1, keepdims=True)
    off2 = jnp.sum(offs * oh2, axis=1, keepdims=True)
    posk_ref[0:S, :] = (off1 + rank1).astype(jnp.int32)
    posk_ref[S:PAIRS, :] = (off2 + rank2).astype(jnp.int32)

    w0b_ref[...] = jnp.broadcast_to(w0, (S, 128))
    w1b_ref[...] = jnp.broadcast_to(w1, (S, 128))

    start_tile = (offs / float(TM)).astype(jnp.int32)
    ntile_tot = jnp.sum(cap) // TM
    t_iota = lax.broadcasted_iota(jnp.int32, (NT, 1), 0)
    t_clamp = jnp.minimum(t_iota, ntile_tot - 1)
    ge = (t_clamp >= start_tile).astype(jnp.int32)
    te = jnp.sum(ge, axis=1, keepdims=True) - 1
    tv = (t_iota < ntile_tot).astype(jnp.int32)

    tef = te.astype(jnp.float32)
    te_shift = jnp.concatenate([tef[0:1, :], tef[:-1, :]], axis=0)
    chg = ((tef != te_shift) & (t_iota > 0)).astype(jnp.float32)
    ri48 = lax.broadcasted_iota(jnp.int32, (NT, NT), 0)
    ci48 = lax.broadcasted_iota(jnp.int32, (NT, NT), 1)
    tri48 = (ri48 >= ci48).astype(jnp.float32)
    run_id = jnp.dot(tri48, chg, preferred_element_type=jnp.float32)
    first = jnp.where(t_iota == 0, 1.0, chg)
    nrun = jnp.sum(chg) + 1.0

    eye48 = (ri48 == ci48).astype(jnp.float32)

    def _row(v):
        return jnp.sum(v * eye48, axis=0, keepdims=True)

    rr = lax.broadcasted_iota(jnp.int32, (NT, 1), 0).astype(jnp.float32)
    mm = ((_row(run_id) == rr) * (_row(first) == 1.0)).astype(jnp.float32)
    er = jnp.dot(mm, tef, preferred_element_type=jnp.float32)
    pf_run = jnp.minimum(run_id + float(RING - 1), nrun - 1.0)
    gg = (_row(rr) == pf_run).astype(jnp.float32)
    pf_e = jnp.dot(gg, er, preferred_element_type=jnp.float32)
    cslot = run_id - RING * jnp.floor(run_id * (1.0 / RING))
    pf_slot = pf_run - RING * jnp.floor(pf_run * (1.0 / RING))
    pf_do = (first * (run_id + float(RING - 1) <= nrun - 1.0)
             * (t_iota > 0))

    ti_c = lax.broadcasted_iota(jnp.int32, (NT, 8), 1)
    t_clampf = t_clamp.astype(jnp.float32)
    ti = jnp.where(ti_c == 0, tef,
         jnp.where(ti_c == 1, tv.astype(jnp.float32),
         jnp.where(ti_c == 2, first,
         jnp.where(ti_c == 3, cslot,
         jnp.where(ti_c == 4, pf_do,
         jnp.where(ti_c == 5, pf_e,
         jnp.where(ti_c == 6, pf_slot, t_clampf)))))))
    ti_ref[...] = ti.astype(jnp.int32)

    def _pick(r):
        idx = jnp.minimum(jnp.float32(r), nrun - 1.0)
        return jnp.sum(er * (rr == idx).astype(jnp.float32))

    hl = lax.broadcasted_iota(jnp.int32, (1, 8), 1)
    hdr = jnp.full((1, 8), 0.0, jnp.float32)
    for r in range(RING):
        hdr = jnp.where(hl == r, _pick(r), hdr)
    for r in range(1, RING):
        hdr = jnp.where(hl == RING + r - 1,
                        (nrun >= float(r + 1)).astype(jnp.float32), hdr)
    hdr_ref[...] = hdr.astype(jnp.int32)


def _router(x_flat, Wr):
    return pl.pallas_call(
        _router_body,
        out_shape=[
            jax.ShapeDtypeStruct((PAIRS, 1), jnp.int32),
            jax.ShapeDtypeStruct((S, 128), jnp.float32),
            jax.ShapeDtypeStruct((S, 128), jnp.float32),
            jax.ShapeDtypeStruct((NT, 8), jnp.int32),
            jax.ShapeDtypeStruct((1, 8), jnp.int32),
        ],
    )(x_flat, Wr)


def _permute_body(x_hbm, posk_hbm, xp_hbm, pos_v, rows_v, sem):
    wid = lax.axis_index("s") * NC + lax.axis_index("c")
    per_w = PAIRS // NW
    ch = 64
    base = pl.multiple_of(wid * per_w, per_w)
    tok_base = pl.multiple_of((wid % NS) * per_w, per_w)
    for i in range(per_w // ch):
        pltpu.sync_copy(posk_hbm.at[pl.ds(base + i * ch, ch)], pos_v)
        pltpu.sync_copy(x_hbm.at[pl.ds(tok_base + i * ch, ch)], rows_v)
        pltpu.async_copy(rows_v, xp_hbm.at[pos_v], sem).wait()


def _sc_permute(x_flat, posk):
    mesh = plsc.VectorSubcoreMesh(core_axis_name="c", subcore_axis_name="s",
                                  num_cores=NC, num_subcores=NS)
    return pl.kernel(
        _permute_body,
        out_type=jax.ShapeDtypeStruct((NSLOT, C), jnp.float32),
        mesh=mesh,
        scratch_types=[
            pltpu.VMEM((64,), jnp.int32),
            pltpu.VMEM((64, C), jnp.float32),
            pltpu.SemaphoreType.DMA,
        ],
    )(x_flat, posk)


def _gemm_body(hdr_ref, ti_ref, x_ref, fc1_any, fc2_any, y_ref,
               w1buf, w2buf, s1, s2):
    t = pl.program_id(0)

    def issue(e, slot):
        pltpu.make_async_copy(fc1_any.at[e], w1buf.at[slot],
                              s1.at[slot]).start()
        pltpu.make_async_copy(fc2_any.at[e], w2buf.at[slot],
                              s2.at[slot]).start()

    def wait(slot):
        pltpu.make_async_copy(fc1_any.at[0], w1buf.at[slot],
                              s1.at[slot]).wait()
        pltpu.make_async_copy(fc2_any.at[0], w2buf.at[slot],
                              s2.at[slot]).wait()

    @pl.when(t == 0)
    def _():
        issue(hdr_ref[0], 0)
        for r in range(1, RING):
            @pl.when(hdr_ref[RING + r - 1] == 1)
            def _(r=r):
                issue(hdr_ref[r], r)

        wait(0)

    @pl.when((t > 0) & (ti_ref[t, 2] == 1))
    def _():
        @pl.when(ti_ref[t, 4] == 1)
        def _():
            issue(ti_ref[t, 5], ti_ref[t, 6])

        wait(ti_ref[t, 3])

    @pl.when(ti_ref[t, 1] == 1)
    def _():
        s = ti_ref[t, 3]
        xt = x_ref[...]
        h = jnp.dot(xt, w1buf[s], preferred_element_type=jnp.float32)
        g, u = h[:, :FFN], h[:, FFN:]
        act = g * (1.0 / (1.0 + jnp.exp(-g))) * u
        y_ref[...] = jnp.dot(act, w2buf[s],
                             preferred_element_type=jnp.float32)


def _grouped_gemm(hdr, ti, x_perm, fc1_w, fc2_w):
    grid_spec = pltpu.PrefetchScalarGridSpec(
        num_scalar_prefetch=2,
        grid=(NT,),
        in_specs=[
            pl.BlockSpec((TM, C), lambda t, hdr, ti: (ti[t, 7], 0)),
            pl.BlockSpec(memory_space=pl.ANY),
            pl.BlockSpec(memory_space=pl.ANY),
        ],
        out_specs=pl.BlockSpec((TM, C), lambda t, hdr, ti: (ti[t, 7], 0)),
        scratch_shapes=[
            pltpu.VMEM((RING, C, 2 * FFN), jnp.float32),
            pltpu.VMEM((RING, FFN, C), jnp.float32),
            pltpu.SemaphoreType.DMA((RING,)),
            pltpu.SemaphoreType.DMA((RING,)),
        ],
    )
    return pl.pallas_call(
        _gemm_body,
        grid_spec=grid_spec,
        out_shape=jax.ShapeDtypeStruct((NSLOT, C), jnp.float32),
        compiler_params=pltpu.CompilerParams(
            dimension_semantics=("arbitrary",)),
    )(hdr, ti, x_perm, fc1_w, fc2_w)


def _unpermute_body(y_hbm, posk_hbm, y0_hbm, y1_hbm, pos_v, rows_v, sem):
    wid = lax.axis_index("s") * NC + lax.axis_index("c")
    per_t = S // NW
    base = pl.multiple_of(wid * per_t, per_t)
    for half, dst in ((0, y0_hbm), (1, y1_hbm)):
        pltpu.sync_copy(posk_hbm.at[pl.ds(half * S + base, per_t)], pos_v)
        pltpu.async_copy(y_hbm.at[pos_v], rows_v, sem).wait()
        pltpu.sync_copy(rows_v, dst.at[pl.ds(base, per_t)])


def _sc_unpermute(y, posk):
    mesh = plsc.VectorSubcoreMesh(core_axis_name="c", subcore_axis_name="s",
                                  num_cores=NC, num_subcores=NS)
    return pl.kernel(
        _unpermute_body,
        out_type=(jax.ShapeDtypeStruct((S, C), jnp.float32),
                  jax.ShapeDtypeStruct((S, C), jnp.float32)),
        mesh=mesh,
        scratch_types=[
            pltpu.VMEM((S // NW,), jnp.int32),
            pltpu.VMEM((S // NW, C), jnp.float32),
            pltpu.SemaphoreType.DMA,
        ],
    )(y, posk)


def _shared_body(x_ref, wg_ref, wu_ref, wd_ref, o_ref):
    xt = x_ref[...]
    g = jnp.dot(xt, wg_ref[...], preferred_element_type=jnp.float32)
    u = jnp.dot(xt, wu_ref[...], preferred_element_type=jnp.float32)
    act = g * (1.0 / (1.0 + jnp.exp(-g))) * u
    o_ref[...] = jnp.dot(act, wd_ref[...], preferred_element_type=jnp.float32)


def _shared(x_flat, Wg_sh, Wu_sh, Wd_sh, half):
    tms = 256
    nt = S // tms // 2
    off = half * nt
    return pl.pallas_call(
        _shared_body,
        grid=(nt,),
        in_specs=[
            pl.BlockSpec((tms, C), lambda t: (t + off, 0)),
            pl.BlockSpec((C, FFN), lambda t: (0, 0)),
            pl.BlockSpec((C, FFN), lambda t: (0, 0)),
            pl.BlockSpec((FFN, C), lambda t: (0, 0)),
        ],
        out_specs=pl.BlockSpec((tms, C), lambda t: (t, 0)),
        out_shape=jax.ShapeDtypeStruct((S // 2, C), jnp.float32),
        compiler_params=pltpu.CompilerParams(
            dimension_semantics=("arbitrary",)),
    )(x_flat, Wg_sh, Wu_sh, Wd_sh)


def _final_body(sh0_ref, sh1_ref, y0_ref, y1_ref, w0_ref, w1_ref, o_ref):
    t = pl.program_id(0)
    sh = jnp.where(t < 4, sh0_ref[...], sh1_ref[...])
    o_ref[...] = (sh + w0_ref[:, 0:1] * y0_ref[...]
                  + w1_ref[:, 0:1] * y1_ref[...])


def _final(sh0, sh1, y0g, y1g, w0b, w1b):
    tms = 256
    return pl.pallas_call(
        _final_body,
        grid=(S // tms,),
        in_specs=[
            pl.BlockSpec((tms, C), lambda t: (jnp.minimum(t, 3), 0)),
            pl.BlockSpec((tms, C), lambda t: (jnp.maximum(t - 4, 0), 0)),
            pl.BlockSpec((tms, C), lambda t: (t, 0)),
            pl.BlockSpec((tms, C), lambda t: (t, 0)),
            pl.BlockSpec((tms, 128), lambda t: (t, 0)),
            pl.BlockSpec((tms, 128), lambda t: (t, 0)),
        ],
        out_specs=pl.BlockSpec((tms, C), lambda t: (t, 0)),
        out_shape=jax.ShapeDtypeStruct((S, C), jnp.float32),
        compiler_params=pltpu.CompilerParams(
            dimension_semantics=("arbitrary",)),
    )(sh0, sh1, y0g, y1g, w0b, w1b)


def kernel(x, Wr, Wg_sh, Wu_sh, Wd_sh, fc1_w, fc2_w):
    x_flat = x.reshape(S, C)
    posk2, w0b, w1b, ti, hdr2 = _router(x_flat, Wr)
    posk = posk2.reshape(PAIRS)
    x_perm = _sc_permute(x_flat, posk)
    sh0 = _shared(x_flat, Wg_sh, Wu_sh, Wd_sh, 0)
    y = _grouped_gemm(hdr2.reshape(8), ti, x_perm, fc1_w, fc2_w)
    sh1 = _shared(x_flat, Wg_sh, Wu_sh, Wd_sh, 1)
    y0g, y1g = _sc_unpermute(y, posk)
    out = _final(sh0, sh1, y0g, y1g, w0b, w1b)
    return out.reshape(B, T, C)


# speedup vs baseline: 1.0543x; 1.0543x over previous
import functools

import jax
import jax.numpy as jnp
from jax import lax
from jax.experimental import pallas as pl
from jax.experimental.pallas import tpu as pltpu
from jax.experimental.pallas import tpu_sc as plsc

B, T, C = 1, 2048, 1024
E, K, FFN = 16, 2, 1024
S = B * T
PAIRS = S * K
TM = 128
NT = PAIRS // TM + E
NSLOT = NT * TM

NC, NS = 2, 16
NW = NC * NS
RING = 3


def _router_body(x_ref, wr_ref, posk_ref, w0b_ref, w1b_ref, ti_ref, hdr_ref):
    x = x_ref[...]
    logits = jnp.dot(x, wr_ref[...], preferred_element_type=jnp.float32)

    eidx = lax.broadcasted_iota(jnp.int32, (S, E), 1)
    m1 = jnp.max(logits, axis=1, keepdims=True)
    a1 = jnp.min(jnp.where(logits == m1, eidx, E), axis=1, keepdims=True)
    masked = jnp.where(eidx == a1, -jnp.inf, logits)
    m2 = jnp.max(masked, axis=1, keepdims=True)
    a2 = jnp.min(jnp.where(masked == m2, eidx, E), axis=1, keepdims=True)

    e2 = jnp.exp(m2 - m1)
    w0 = 1.0 / (1.0 + e2)
    w1 = e2 / (1.0 + e2)

    oh1 = (eidx == a1).astype(jnp.float32)
    oh2 = (eidx == a2).astype(jnp.float32)
    oh = jnp.concatenate([oh1, oh2], axis=1)

    ch = 256
    ri = lax.broadcasted_iota(jnp.int32, (ch, ch), 0)
    ci = lax.broadcasted_iota(jnp.int32, (ch, ch), 1)
    tri = (ri > ci).astype(jnp.float32)
    carry = jnp.zeros((1, 2 * E), jnp.float32)
    parts = []
    for b in range(S // ch):
        blk = oh[b * ch:(b + 1) * ch, :]
        parts.append(jnp.dot(tri, blk, preferred_element_type=jnp.float32)
                     + carry)
        carry = carry + jnp.sum(blk, axis=0, keepdims=True)
    csum = jnp.concatenate(parts, axis=0)

    cnt = carry
    cnt1, cnt2 = cnt[:, :E], cnt[:, E:]
    cnt_tot = (cnt1 + cnt2).astype(jnp.int32)
    cap = ((cnt_tot + (TM - 1)) // TM) * TM

    eli = lax.broadcasted_iota(jnp.int32, (E, E), 0)
    eci = lax.broadcasted_iota(jnp.int32, (E, E), 1)
    lt = (eli < eci).astype(jnp.float32)
    offs = jnp.dot(cap.astype(jnp.float32), lt,
                   preferred_element_type=jnp.float32)

    rank1 = jnp.sum(csum[:, :E] * oh1, axis=1, keepdims=True)
    rank2 = jnp.sum((csum[:, E:] + cnt1) * oh2, axis=1, keepdims=True)
    off1 = jnp.sum(offs * oh1, axis=1, keepdims=True)
    off2 = jnp.sum(offs * oh2, axis=1, keepdims=True)
    posk_ref[0:S, :] = (off1 + rank1).astype(jnp.int32)
    posk_ref[S:PAIRS, :] = (off2 + rank2).astype(jnp.int32)

    w0b_ref[...] = jnp.broadcast_to(w0, (S, 128))
    w1b_ref[...] = jnp.broadcast_to(w1, (S, 128))

    start_tile = (offs / float(TM)).astype(jnp.int32)
    ntile_tot = jnp.sum(cap) // TM
    t_iota = lax.broadcasted_iota(jnp.int32, (NT, 1), 0)
    t_clamp = jnp.minimum(t_iota, ntile_tot - 1)
    ge = (t_clamp >= start_tile).astype(jnp.int32)
    te = jnp.sum(ge, axis=1, keepdims=True) - 1
    tv = (t_iota < ntile_tot).astype(jnp.int32)

    tef = te.astype(jnp.float32)
    te_shift = jnp.concatenate([tef[0:1, :], tef[:-1, :]], axis=0)
    chg = ((tef != te_shift) & (t_iota > 0)).astype(jnp.float32)
    ri48 = lax.broadcasted_iota(jnp.int32, (NT, NT), 0)
    ci48 = lax.broadcasted_iota(jnp.int32, (NT, NT), 1)
    tri48 = (ri48 >= ci48).astype(jnp.float32)
    run_id = jnp.dot(tri48, chg, preferred_element_type=jnp.float32)
    first = jnp.where(t_iota == 0, 1.0, chg)
    nrun = jnp.sum(chg) + 1.0

    eye48 = (ri48 == ci48).astype(jnp.float32)

    def _row(v):
        return jnp.sum(v * eye48, axis=0, keepdims=True)

    rr = lax.broadcasted_iota(jnp.int32, (NT, 1), 0).astype(jnp.float32)
    mm = ((_row(run_id) == rr) * (_row(first) == 1.0)).astype(jnp.float32)
    er = jnp.dot(mm, tef, preferred_element_type=jnp.float32)
    pf_run = jnp.minimum(run_id + float(RING - 1), nrun - 1.0)
    gg = (_row(rr) == pf_run).astype(jnp.float32)
    pf_e = jnp.dot(gg, er, preferred_element_type=jnp.float32)
    cslot = run_id - RING * jnp.floor(run_id * (1.0 / RING))
    pf_slot = pf_run - RING * jnp.floor(pf_run * (1.0 / RING))
    pf_do = (first * (run_id + float(RING - 1) <= nrun - 1.0)
             * (t_iota > 0))

    ti_c = lax.broadcasted_iota(jnp.int32, (NT, 8), 1)
    t_clampf = t_clamp.astype(jnp.float32)
    ti = jnp.where(ti_c == 0, tef,
         jnp.where(ti_c == 1, tv.astype(jnp.float32),
         jnp.where(ti_c == 2, first,
         jnp.where(ti_c == 3, cslot,
         jnp.where(ti_c == 4, pf_do,
         jnp.where(ti_c == 5, pf_e,
         jnp.where(ti_c == 6, pf_slot, t_clampf)))))))
    ti_ref[...] = ti.astype(jnp.int32)

    def _pick(r):
        idx = jnp.minimum(jnp.float32(r), nrun - 1.0)
        return jnp.sum(er * (rr == idx).astype(jnp.float32))

    hl = lax.broadcasted_iota(jnp.int32, (1, 8), 1)
    hdr = jnp.full((1, 8), 0.0, jnp.float32)
    for r in range(RING):
        hdr = jnp.where(hl == r, _pick(r), hdr)
    for r in range(1, RING):
        hdr = jnp.where(hl == RING + r - 1,
                        (nrun >= float(r + 1)).astype(jnp.float32), hdr)
    hdr_ref[...] = hdr.astype(jnp.int32)


def _router(x_flat, Wr):
    return pl.pallas_call(
        _router_body,
        out_shape=[
            jax.ShapeDtypeStruct((PAIRS, 1), jnp.int32),
            jax.ShapeDtypeStruct((S, 128), jnp.float32),
            jax.ShapeDtypeStruct((S, 128), jnp.float32),
            jax.ShapeDtypeStruct((NT, 8), jnp.int32),
            jax.ShapeDtypeStruct((1, 8), jnp.int32),
        ],
    )(x_flat, Wr)


def _permute_body(x_hbm, posk_hbm, xp_hbm, pos_v, rows_v, sem):
    wid = lax.axis_index("s") * NC + lax.axis_index("c")
    per_w = PAIRS // NW
    ch = 64
    base = pl.multiple_of(wid * per_w, per_w)
    tok_base = pl.multiple_of((wid % NS) * per_w, per_w)
    for i in range(per_w // ch):
        pltpu.sync_copy(posk_hbm.at[pl.ds(base + i * ch, ch)], pos_v)
        pltpu.sync_copy(x_hbm.at[pl.ds(tok_base + i * ch, ch)], rows_v)
        pltpu.async_copy(rows_v, xp_hbm.at[pos_v], sem).wait()


def _sc_permute(x_flat, posk):
    mesh = plsc.VectorSubcoreMesh(core_axis_name="c", subcore_axis_name="s",
                                  num_cores=NC, num_subcores=NS)
    return pl.kernel(
        _permute_body,
        out_type=jax.ShapeDtypeStruct((NSLOT, C), jnp.float32),
        mesh=mesh,
        scratch_types=[
            pltpu.VMEM((64,), jnp.int32),
            pltpu.VMEM((64, C), jnp.float32),
            pltpu.SemaphoreType.DMA,
        ],
    )(x_flat, posk)


def _gemm_body(hdr_ref, ti_ref, x_ref, fc1_any, fc2_any, y_ref,
               w1buf, w2buf, s1, s2):
    t = pl.program_id(0)

    def issue(e, slot):
        pltpu.make_async_copy(fc1_any.at[e], w1buf.at[slot],
                              s1.at[slot]).start()
        pltpu.make_async_copy(fc2_any.at[e], w2buf.at[slot],
                              s2.at[slot]).start()

    def wait(slot):
        pltpu.make_async_copy(fc1_any.at[0], w1buf.at[slot],
                              s1.at[slot]).wait()
        pltpu.make_async_copy(fc2_any.at[0], w2buf.at[slot],
                              s2.at[slot]).wait()

    @pl.when(t == 0)
    def _():
        issue(hdr_ref[0], 0)
        for r in range(1, RING):
            @pl.when(hdr_ref[RING + r - 1] == 1)
            def _(r=r):
                issue(hdr_ref[r], r)

        wait(0)

    @pl.when((t > 0) & (ti_ref[t, 2] == 1))
    def _():
        @pl.when(ti_ref[t, 4] == 1)
        def _():
            issue(ti_ref[t, 5], ti_ref[t, 6])

        wait(ti_ref[t, 3])

    @pl.when(ti_ref[t, 1] == 1)
    def _():
        s = ti_ref[t, 3]
        xt = x_ref[...]
        h = jnp.dot(xt, w1buf[s], preferred_element_type=jnp.float32)
        g, u = h[:, :FFN], h[:, FFN:]
        act = g * (1.0 / (1.0 + jnp.exp(-g))) * u
        y_ref[...] = jnp.dot(act, w2buf[s],
                             preferred_element_type=jnp.float32)


def _grouped_gemm(hdr, ti, x_perm, fc1_w, fc2_w):
    grid_spec = pltpu.PrefetchScalarGridSpec(
        num_scalar_prefetch=2,
        grid=(NT,),
        in_specs=[
            pl.BlockSpec((TM, C), lambda t, hdr, ti: (ti[t, 7], 0)),
            pl.BlockSpec(memory_space=pl.ANY),
            pl.BlockSpec(memory_space=pl.ANY),
        ],
        out_specs=pl.BlockSpec((TM, C), lambda t, hdr, ti: (ti[t, 7], 0)),
        scratch_shapes=[
            pltpu.VMEM((RING, C, 2 * FFN), jnp.float32),
            pltpu.VMEM((RING, FFN, C), jnp.float32),
            pltpu.SemaphoreType.DMA((RING,)),
            pltpu.SemaphoreType.DMA((RING,)),
        ],
    )
    return pl.pallas_call(
        _gemm_body,
        grid_spec=grid_spec,
        out_shape=jax.ShapeDtypeStruct((NSLOT, C), jnp.float32),
        compiler_params=pltpu.CompilerParams(
            dimension_semantics=("arbitrary",)),
    )(hdr, ti, x_perm, fc1_w, fc2_w)


def _unpermute_body(y_hbm, posk_hbm, y0_hbm, y1_hbm, pos_v, rows_v, sem):
    wid = lax.axis_index("s") * NC + lax.axis_index("c")
    per_t = S // NW
    base = pl.multiple_of(wid * per_t, per_t)
    for half, dst in ((0, y0_hbm), (1, y1_hbm)):
        pltpu.sync_copy(posk_hbm.at[pl.ds(half * S + base, per_t)], pos_v)
        pltpu.async_copy(y_hbm.at[pos_v], rows_v, sem).wait()
        pltpu.sync_copy(rows_v, dst.at[pl.ds(base, per_t)])


def _sc_unpermute(y, posk):
    mesh = plsc.VectorSubcoreMesh(core_axis_name="c", subcore_axis_name="s",
                                  num_cores=NC, num_subcores=NS)
    return pl.kernel(
        _unpermute_body,
        out_type=(jax.ShapeDtypeStruct((S, C), jnp.float32),
                  jax.ShapeDtypeStruct((S, C), jnp.float32)),
        mesh=mesh,
        scratch_types=[
            pltpu.VMEM((S // NW,), jnp.int32),
            pltpu.VMEM((S // NW, C), jnp.float32),
            pltpu.SemaphoreType.DMA,
        ],
    )(y, posk)


def _shared_body(x_ref, wg_ref, wu_ref, wd_ref, o_ref):
    xt = x_ref[...]
    g = jnp.dot(xt, wg_ref[...], preferred_element_type=jnp.float32)
    u = jnp.dot(xt, wu_ref[...], preferred_element_type=jnp.float32)
    act = g * (1.0 / (1.0 + jnp.exp(-g))) * u
    o_ref[...] = jnp.dot(act, wd_ref[...], preferred_element_type=jnp.float32)


def _shared(x_flat, Wg_sh, Wu_sh, Wd_sh, half):
    tms = 256
    nt = S // tms // 2
    off = half * nt
    return pl.pallas_call(
        _shared_body,
        grid=(nt,),
        in_specs=[
            pl.BlockSpec((tms, C), lambda t: (t + off, 0)),
            pl.BlockSpec((C, FFN), lambda t: (0, 0)),
            pl.BlockSpec((C, FFN), lambda t: (0, 0)),
            pl.BlockSpec((FFN, C), lambda t: (0, 0)),
        ],
        out_specs=pl.BlockSpec((tms, C), lambda t: (t, 0)),
        out_shape=jax.ShapeDtypeStruct((S // 2, C), jnp.float32),
        compiler_params=pltpu.CompilerParams(
            dimension_semantics=("arbitrary",)),
    )(x_flat, Wg_sh, Wu_sh, Wd_sh)


def _final_body(sh0_ref, sh1_ref, y0_ref, y1_ref, w0_ref, w1_ref, o_ref):
    t = pl.program_id(0)
    sh = jnp.where(t < 4, sh0_ref[...], sh1_ref[...])
    o_ref[...] = (sh + w0_ref[:, 0:1] * y0_ref[...]
                  + w1_ref[:, 0:1] * y1_ref[...])


def _final(sh0, sh1, y0g, y1g, w0b, w1b):
    tms = 256
    return pl.pallas_call(
        _final_body,
        grid=(S // tms,),
        in_specs=[
            pl.BlockSpec((tms, C), lambda t: (jnp.minimum(t, 3), 0)),
            pl.BlockSpec((tms, C), lambda t: (jnp.maximum(t - 4, 0), 0)),
            pl.BlockSpec((tms, C), lambda t: (t, 0)),
            pl.BlockSpec((tms, C), lambda t: (t, 0)),
            pl.BlockSpec((tms, 128), lambda t: (t, 0)),
            pl.BlockSpec((tms, 128), lambda t: (t, 0)),
        ],
        out_specs=pl.BlockSpec((tms, C), lambda t: (t, 0)),
        out_shape=jax.ShapeDtypeStruct((S, C), jnp.float32),
        compiler_params=pltpu.CompilerParams(
            dimension_semantics=("arbitrary",)),
    )(sh0, sh1, y0g, y1g, w0b, w1b)


def kernel(x, Wr, Wg_sh, Wu_sh, Wd_sh, fc1_w, fc2_w):
    x_flat = x.reshape(S, C)
    posk2, w0b, w1b, ti, hdr2 = _router(x_flat, Wr)
    posk = posk2.reshape(PAIRS)
    x_perm = _sc_permute(x_flat, posk)
    sh0 = _shared(x_flat, Wg_sh, Wu_sh, Wd_sh, 0)
    dep = (sh0[0, 0] * 0.0).astype(jnp.int32)
    y = _grouped_gemm(hdr2.reshape(8) + dep, ti, x_perm, fc1_w, fc2_w)
    sh1 = _shared(x_flat, Wg_sh, Wu_sh, Wd_sh, 1)
    y0g, y1g = _sc_unpermute(y, posk)
    out = _final(sh0, sh1, y0g, y1g, w0b, w1b)
    return out.reshape(B, T, C)


# speedup vs baseline: 1.0952x; 1.0388x over previous
import functools

import jax
import jax.numpy as jnp
from jax import lax
from jax.experimental import pallas as pl
from jax.experimental.pallas import tpu as pltpu
from jax.experimental.pallas import tpu_sc as plsc

B, T, C = 1, 2048, 1024
E, K, FFN = 16, 2, 1024
S = B * T
PAIRS = S * K
TM = 128
NT = PAIRS // TM + E
NSLOT = NT * TM

NC, NS = 2, 16
NW = NC * NS
RING = 3


def _router_body(x_ref, wr_ref, posk_ref, w0b_ref, w1b_ref, ti_ref, hdr_ref):
    x = x_ref[...]
    logits = jnp.dot(x, wr_ref[...], preferred_element_type=jnp.float32)

    eidx = lax.broadcasted_iota(jnp.int32, (S, E), 1)
    m1 = jnp.max(logits, axis=1, keepdims=True)
    a1 = jnp.min(jnp.where(logits == m1, eidx, E), axis=1, keepdims=True)
    masked = jnp.where(eidx == a1, -jnp.inf, logits)
    m2 = jnp.max(masked, axis=1, keepdims=True)
    a2 = jnp.min(jnp.where(masked == m2, eidx, E), axis=1, keepdims=True)

    e2 = jnp.exp(m2 - m1)
    w0 = 1.0 / (1.0 + e2)
    w1 = e2 / (1.0 + e2)

    oh1 = (eidx == a1).astype(jnp.float32)
    oh2 = (eidx == a2).astype(jnp.float32)
    oh = jnp.concatenate([oh1, oh2], axis=1)

    ch = 256
    ri = lax.broadcasted_iota(jnp.int32, (ch, ch), 0)
    ci = lax.broadcasted_iota(jnp.int32, (ch, ch), 1)
    tri = (ri > ci).astype(jnp.float32)
    carry = jnp.zeros((1, 2 * E), jnp.float32)
    parts = []
    for b in range(S // ch):
        blk = oh[b * ch:(b + 1) * ch, :]
        parts.append(jnp.dot(tri, blk, preferred_element_type=jnp.float32)
                     + carry)
        carry = carry + jnp.sum(blk, axis=0, keepdims=True)
    csum = jnp.concatenate(parts, axis=0)

    cnt = carry
    cnt1, cnt2 = cnt[:, :E], cnt[:, E:]
    cnt_tot = (cnt1 + cnt2).astype(jnp.int32)
    cap = ((cnt_tot + (TM - 1)) // TM) * TM

    eli = lax.broadcasted_iota(jnp.int32, (E, E), 0)
    eci = lax.broadcasted_iota(jnp.int32, (E, E), 1)
    lt = (eli < eci).astype(jnp.float32)
    offs = jnp.dot(cap.astype(jnp.float32), lt,
                   preferred_element_type=jnp.float32)

    rank1 = jnp.sum(csum[:, :E] * oh1, axis=1, keepdims=True)
    rank2 = jnp.sum((csum[:, E:] + cnt1) * oh2, axis=1, keepdims=True)
    off1 = jnp.sum(offs * oh1, axis=1, keepdims=True)
    off2 = jnp.sum(offs * oh2, axis=1, keepdims=True)
    posk_ref[0:S, :] = (off1 + rank1).astype(jnp.int32)
    posk_ref[S:PAIRS, :] = (off2 + rank2).astype(jnp.int32)

    w0b_ref[...] = jnp.broadcast_to(w0, (S, 128))
    w1b_ref[...] = jnp.broadcast_to(w1, (S, 128))

    start_tile = (offs / float(TM)).astype(jnp.int32)
    ntile_tot = jnp.sum(cap) // TM
    t_iota = lax.broadcasted_iota(jnp.int32, (NT, 1), 0)
    t_clamp = jnp.minimum(t_iota, ntile_tot - 1)
    ge = (t_clamp >= start_tile).astype(jnp.int32)
    te = jnp.sum(ge, axis=1, keepdims=True) - 1
    tv = (t_iota < ntile_tot).astype(jnp.int32)

    tef = te.astype(jnp.float32)
    te_shift = jnp.concatenate([tef[0:1, :], tef[:-1, :]], axis=0)
    chg = ((tef != te_shift) & (t_iota > 0)).astype(jnp.float32)
    ri48 = lax.broadcasted_iota(jnp.int32, (NT, NT), 0)
    ci48 = lax.broadcasted_iota(jnp.int32, (NT, NT), 1)
    tri48 = (ri48 >= ci48).astype(jnp.float32)
    run_id = jnp.dot(tri48, chg, preferred_element_type=jnp.float32)
    first = jnp.where(t_iota == 0, 1.0, chg)
    nrun = jnp.sum(chg) + 1.0

    eye48 = (ri48 == ci48).astype(jnp.float32)

    def _row(v):
        return jnp.sum(v * eye48, axis=0, keepdims=True)

    rr = lax.broadcasted_iota(jnp.int32, (NT, 1), 0).astype(jnp.float32)
    mm = ((_row(run_id) == rr) * (_row(first) == 1.0)).astype(jnp.float32)
    er = jnp.dot(mm, tef, preferred_element_type=jnp.float32)
    pf_run = jnp.minimum(run_id + float(RING - 1), nrun - 1.0)
    gg = (_row(rr) == pf_run).astype(jnp.float32)
    pf_e = jnp.dot(gg, er, preferred_element_type=jnp.float32)
    cslot = run_id - RING * jnp.floor(run_id * (1.0 / RING))
    pf_slot = pf_run - RING * jnp.floor(pf_run * (1.0 / RING))
    pf_do = (first * (run_id + float(RING - 1) <= nrun - 1.0)
             * (t_iota > 0))

    ti_c = lax.broadcasted_iota(jnp.int32, (NT, 8), 1)
    t_clampf = t_clamp.astype(jnp.float32)
    ti = jnp.where(ti_c == 0, tef,
         jnp.where(ti_c == 1, tv.astype(jnp.float32),
         jnp.where(ti_c == 2, first,
         jnp.where(ti_c == 3, cslot,
         jnp.where(ti_c == 4, pf_do,
         jnp.where(ti_c == 5, pf_e,
         jnp.where(ti_c == 6, pf_slot, t_clampf)))))))
    ti_ref[...] = ti.astype(jnp.int32)

    def _pick(r):
        idx = jnp.minimum(jnp.float32(r), nrun - 1.0)
        return jnp.sum(er * (rr == idx).astype(jnp.float32))

    hl = lax.broadcasted_iota(jnp.int32, (1, 8), 1)
    hdr = jnp.full((1, 8), 0.0, jnp.float32)
    for r in range(RING):
        hdr = jnp.where(hl == r, _pick(r), hdr)
    for r in range(1, RING):
        hdr = jnp.where(hl == RING + r - 1,
                        (nrun >= float(r + 1)).astype(jnp.float32), hdr)
    hdr_ref[...] = hdr.astype(jnp.int32)


def _router(x_flat, Wr):
    return pl.pallas_call(
        _router_body,
        out_shape=[
            jax.ShapeDtypeStruct((PAIRS, 1), jnp.int32),
            jax.ShapeDtypeStruct((S, 128), jnp.float32),
            jax.ShapeDtypeStruct((S, 128), jnp.float32),
            jax.ShapeDtypeStruct((NT, 8), jnp.int32),
            jax.ShapeDtypeStruct((1, 8), jnp.int32),
        ],
    )(x_flat, Wr)


def _permute_body(x_hbm, posk_hbm, xp_hbm, pos_v, rows_v, sem):
    wid = lax.axis_index("s") * NC + lax.axis_index("c")
    per_w = PAIRS // NW
    ch = 64
    base = pl.multiple_of(wid * per_w, per_w)
    tok_base = pl.multiple_of((wid % NS) * per_w, per_w)
    for i in range(per_w // ch):
        pltpu.sync_copy(posk_hbm.at[pl.ds(base + i * ch, ch)], pos_v)
        pltpu.sync_copy(x_hbm.at[pl.ds(tok_base + i * ch, ch)], rows_v)
        pltpu.async_copy(rows_v, xp_hbm.at[pos_v], sem).wait()


def _sc_permute(x_flat, posk):
    mesh = plsc.VectorSubcoreMesh(core_axis_name="c", subcore_axis_name="s",
                                  num_cores=NC, num_subcores=NS)
    return pl.kernel(
        _permute_body,
        out_type=jax.ShapeDtypeStruct((NSLOT, C), jnp.float32),
        mesh=mesh,
        scratch_types=[
            pltpu.VMEM((64,), jnp.int32),
            pltpu.VMEM((64, C), jnp.float32),
            pltpu.SemaphoreType.DMA,
        ],
    )(x_flat, posk)


def _gemm_body(hdr_ref, ti_ref, x_ref, fc1_any, fc2_any, y_ref,
               w1buf, w2buf, s1, s2):
    t = pl.program_id(0)

    def issue(e, slot):
        pltpu.make_async_copy(fc1_any.at[e], w1buf.at[slot],
                              s1.at[slot]).start()
        pltpu.make_async_copy(fc2_any.at[e], w2buf.at[slot],
                              s2.at[slot]).start()

    def wait(slot):
        pltpu.make_async_copy(fc1_any.at[0], w1buf.at[slot],
                              s1.at[slot]).wait()
        pltpu.make_async_copy(fc2_any.at[0], w2buf.at[slot],
                              s2.at[slot]).wait()

    @pl.when(t == 0)
    def _():
        issue(hdr_ref[0], 0)
        for r in range(1, RING):
            @pl.when(hdr_ref[RING + r - 1] == 1)
            def _(r=r):
                issue(hdr_ref[r], r)

        wait(0)

    @pl.when((t > 0) & (ti_ref[t, 2] == 1))
    def _():
        @pl.when(ti_ref[t, 4] == 1)
        def _():
            issue(ti_ref[t, 5], ti_ref[t, 6])

        wait(ti_ref[t, 3])

    @pl.when(ti_ref[t, 1] == 1)
    def _():
        s = ti_ref[t, 3]
        xt = x_ref[...]
        h = jnp.dot(xt, w1buf[s], preferred_element_type=jnp.float32)
        g, u = h[:, :FFN], h[:, FFN:]
        act = g * (1.0 / (1.0 + jnp.exp(-g))) * u
        yf = jnp.dot(act, w2buf[s], preferred_element_type=jnp.float32)
        y_ref[...] = pltpu.pack_elementwise(
            [yf[:, :C // 2], yf[:, C // 2:]], packed_dtype=jnp.bfloat16)


def _grouped_gemm(hdr, ti, x_perm, fc1_w, fc2_w):
    grid_spec = pltpu.PrefetchScalarGridSpec(
        num_scalar_prefetch=2,
        grid=(NT,),
        in_specs=[
            pl.BlockSpec((TM, C), lambda t, hdr, ti: (ti[t, 7], 0)),
            pl.BlockSpec(memory_space=pl.ANY),
            pl.BlockSpec(memory_space=pl.ANY),
        ],
        out_specs=pl.BlockSpec((TM, C // 2),
                               lambda t, hdr, ti: (ti[t, 7], 0)),
        scratch_shapes=[
            pltpu.VMEM((RING, C, 2 * FFN), jnp.float32),
            pltpu.VMEM((RING, FFN, C), jnp.float32),
            pltpu.SemaphoreType.DMA((RING,)),
            pltpu.SemaphoreType.DMA((RING,)),
        ],
    )
    return pl.pallas_call(
        _gemm_body,
        grid_spec=grid_spec,
        out_shape=jax.ShapeDtypeStruct((NSLOT, C // 2), jnp.int32),
        compiler_params=pltpu.CompilerParams(
            dimension_semantics=("arbitrary",)),
    )(hdr, ti, x_perm, fc1_w, fc2_w)


def _unpermute_body(y_hbm, posk_hbm, y0_hbm, y1_hbm, pos_v, rows_v, sem):
    wid = lax.axis_index("s") * NC + lax.axis_index("c")
    per_t = S // NW
    base = pl.multiple_of(wid * per_t, per_t)
    for half, dst in ((0, y0_hbm), (1, y1_hbm)):
        pltpu.sync_copy(posk_hbm.at[pl.ds(half * S + base, per_t)], pos_v)
        pltpu.async_copy(y_hbm.at[pos_v], rows_v, sem).wait()
        pltpu.sync_copy(rows_v, dst.at[pl.ds(base, per_t)])


def _sc_unpermute(y, posk):
    mesh = plsc.VectorSubcoreMesh(core_axis_name="c", subcore_axis_name="s",
                                  num_cores=NC, num_subcores=NS)
    return pl.kernel(
        _unpermute_body,
        out_type=(jax.ShapeDtypeStruct((S, C // 2), jnp.int32),
                  jax.ShapeDtypeStruct((S, C // 2), jnp.int32)),
        mesh=mesh,
        scratch_types=[
            pltpu.VMEM((S // NW,), jnp.int32),
            pltpu.VMEM((S // NW, C // 2), jnp.int32),
            pltpu.SemaphoreType.DMA,
        ],
    )(y, posk)


def _shared_body(x_ref, wg_ref, wu_ref, wd_ref, o_ref):
    xt = x_ref[...]
    g = jnp.dot(xt, wg_ref[...], preferred_element_type=jnp.float32)
    u = jnp.dot(xt, wu_ref[...], preferred_element_type=jnp.float32)
    act = g * (1.0 / (1.0 + jnp.exp(-g))) * u
    o_ref[...] = jnp.dot(act, wd_ref[...], preferred_element_type=jnp.float32)


def _shared(x_flat, Wg_sh, Wu_sh, Wd_sh, half):
    tms = 256
    nt = S // tms // 2
    off = half * nt
    return pl.pallas_call(
        _shared_body,
        grid=(nt,),
        in_specs=[
            pl.BlockSpec((tms, C), lambda t: (t + off, 0)),
            pl.BlockSpec((C, FFN), lambda t: (0, 0)),
            pl.BlockSpec((C, FFN), lambda t: (0, 0)),
            pl.BlockSpec((FFN, C), lambda t: (0, 0)),
        ],
        out_specs=pl.BlockSpec((tms, C), lambda t: (t, 0)),
        out_shape=jax.ShapeDtypeStruct((S // 2, C), jnp.float32),
        compiler_params=pltpu.CompilerParams(
            dimension_semantics=("arbitrary",)),
    )(x_flat, Wg_sh, Wu_sh, Wd_sh)


def _unpack_bf16(yi):
    lo = pltpu.unpack_elementwise(yi, index=0, packed_dtype=jnp.bfloat16,
                                  unpacked_dtype=jnp.float32)
    hi = pltpu.unpack_elementwise(yi, index=1, packed_dtype=jnp.bfloat16,
                                  unpacked_dtype=jnp.float32)
    return jnp.concatenate([lo, hi], axis=1)


def _final_body(sh0_ref, sh1_ref, y0_ref, y1_ref, w0_ref, w1_ref, o_ref):
    t = pl.program_id(0)
    sh = jnp.where(t < 4, sh0_ref[...], sh1_ref[...])
    o_ref[...] = (sh + w0_ref[:, 0:1] * _unpack_bf16(y0_ref[...])
                  + w1_ref[:, 0:1] * _unpack_bf16(y1_ref[...]))


def _final(sh0, sh1, y0g, y1g, w0b, w1b):
    tms = 256
    return pl.pallas_call(
        _final_body,
        grid=(S // tms,),
        in_specs=[
            pl.BlockSpec((tms, C), lambda t: (jnp.minimum(t, 3), 0)),
            pl.BlockSpec((tms, C), lambda t: (jnp.maximum(t - 4, 0), 0)),
            pl.BlockSpec((tms, C // 2), lambda t: (t, 0)),
            pl.BlockSpec((tms, C // 2), lambda t: (t, 0)),
            pl.BlockSpec((tms, 128), lambda t: (t, 0)),
            pl.BlockSpec((tms, 128), lambda t: (t, 0)),
        ],
        out_specs=pl.BlockSpec((tms, C), lambda t: (t, 0)),
        out_shape=jax.ShapeDtypeStruct((S, C), jnp.float32),
        compiler_params=pltpu.CompilerParams(
            dimension_semantics=("arbitrary",)),
    )(sh0, sh1, y0g, y1g, w0b, w1b)


def kernel(x, Wr, Wg_sh, Wu_sh, Wd_sh, fc1_w, fc2_w):
    x_flat = x.reshape(S, C)
    posk2, w0b, w1b, ti, hdr2 = _router(x_flat, Wr)
    posk = posk2.reshape(PAIRS)
    x_perm = _sc_permute(x_flat, posk)
    sh0 = _shared(x_flat, Wg_sh, Wu_sh, Wd_sh, 0)
    dep = (sh0[0, 0] * 0.0).astype(jnp.int32)
    y = _grouped_gemm(hdr2.reshape(8) + dep, ti, x_perm, fc1_w, fc2_w)
    sh1 = _shared(x_flat, Wg_sh, Wu_sh, Wd_sh, 1)
    y0g, y1g = _sc_unpermute(y, posk)
    out = _final(sh0, sh1, y0g, y1g, w0b, w1b)
    return out.reshape(B, T, C)


# speedup vs baseline: 1.1145x; 1.0177x over previous
import functools

import jax
import jax.numpy as jnp
from jax import lax
from jax.experimental import pallas as pl
from jax.experimental.pallas import tpu as pltpu
from jax.experimental.pallas import tpu_sc as plsc

B, T, C = 1, 2048, 1024
E, K, FFN = 16, 2, 1024
S = B * T
PAIRS = S * K
TM = 128
NT = PAIRS // TM + E
NSLOT = NT * TM

NC, NS = 2, 16
NW = NC * NS
RING = 3


def _router_body(x_ref, wr_ref, posk_ref, w0b_ref, w1b_ref, ti_ref, hdr_ref,
                 xp16_ref):
    x = x_ref[...]
    xp16_ref[...] = pltpu.pack_elementwise(
        [x[:, :C // 2], x[:, C // 2:]], packed_dtype=jnp.bfloat16)
    logits = jnp.dot(x, wr_ref[...], preferred_element_type=jnp.float32)

    eidx = lax.broadcasted_iota(jnp.int32, (S, E), 1)
    m1 = jnp.max(logits, axis=1, keepdims=True)
    a1 = jnp.min(jnp.where(logits == m1, eidx, E), axis=1, keepdims=True)
    masked = jnp.where(eidx == a1, -jnp.inf, logits)
    m2 = jnp.max(masked, axis=1, keepdims=True)
    a2 = jnp.min(jnp.where(masked == m2, eidx, E), axis=1, keepdims=True)

    e2 = jnp.exp(m2 - m1)
    w0 = 1.0 / (1.0 + e2)
    w1 = e2 / (1.0 + e2)

    oh1 = (eidx == a1).astype(jnp.float32)
    oh2 = (eidx == a2).astype(jnp.float32)
    oh = jnp.concatenate([oh1, oh2], axis=1)

    ch = 256
    ri = lax.broadcasted_iota(jnp.int32, (ch, ch), 0)
    ci = lax.broadcasted_iota(jnp.int32, (ch, ch), 1)
    tri = (ri > ci).astype(jnp.float32)
    carry = jnp.zeros((1, 2 * E), jnp.float32)
    parts = []
    for b in range(S // ch):
        blk = oh[b * ch:(b + 1) * ch, :]
        parts.append(jnp.dot(tri, blk, preferred_element_type=jnp.float32)
                     + carry)
        carry = carry + jnp.sum(blk, axis=0, keepdims=True)
    csum = jnp.concatenate(parts, axis=0)

    cnt = carry
    cnt1, cnt2 = cnt[:, :E], cnt[:, E:]
    cnt_tot = (cnt1 + cnt2).astype(jnp.int32)
    cap = ((cnt_tot + (TM - 1)) // TM) * TM

    eli = lax.broadcasted_iota(jnp.int32, (E, E), 0)
    eci = lax.broadcasted_iota(jnp.int32, (E, E), 1)
    lt = (eli < eci).astype(jnp.float32)
    offs = jnp.dot(cap.astype(jnp.float32), lt,
                   preferred_element_type=jnp.float32)

    rank1 = jnp.sum(csum[:, :E] * oh1, axis=1, keepdims=True)
    rank2 = jnp.sum((csum[:, E:] + cnt1) * oh2, axis=1, keepdims=True)
    off1 = jnp.sum(offs * oh1, axis=1, keepdims=True)
    off2 = jnp.sum(offs * oh2, axis=1, keepdims=True)
    posk_ref[0:S, :] = (off1 + rank1).astype(jnp.int32)
    posk_ref[S:PAIRS, :] = (off2 + rank2).astype(jnp.int32)

    w0b_ref[...] = jnp.broadcast_to(w0, (S, 128))
    w1b_ref[...] = jnp.broadcast_to(w1, (S, 128))

    start_tile = (offs / float(TM)).astype(jnp.int32)
    ntile_tot = jnp.sum(cap) // TM
    t_iota = lax.broadcasted_iota(jnp.int32, (NT, 1), 0)
    t_clamp = jnp.minimum(t_iota, ntile_tot - 1)
    ge = (t_clamp >= start_tile).astype(jnp.int32)
    te = jnp.sum(ge, axis=1, keepdims=True) - 1
    tv = (t_iota < ntile_tot).astype(jnp.int32)

    tef = te.astype(jnp.float32)
    te_shift = jnp.concatenate([tef[0:1, :], tef[:-1, :]], axis=0)
    chg = ((tef != te_shift) & (t_iota > 0)).astype(jnp.float32)
    ri48 = lax.broadcasted_iota(jnp.int32, (NT, NT), 0)
    ci48 = lax.broadcasted_iota(jnp.int32, (NT, NT), 1)
    tri48 = (ri48 >= ci48).astype(jnp.float32)
    run_id = jnp.dot(tri48, chg, preferred_element_type=jnp.float32)
    first = jnp.where(t_iota == 0, 1.0, chg)
    nrun = jnp.sum(chg) + 1.0

    eye48 = (ri48 == ci48).astype(jnp.float32)

    def _row(v):
        return jnp.sum(v * eye48, axis=0, keepdims=True)

    rr = lax.broadcasted_iota(jnp.int32, (NT, 1), 0).astype(jnp.float32)
    mm = ((_row(run_id) == rr) * (_row(first) == 1.0)).astype(jnp.float32)
    er = jnp.dot(mm, tef, preferred_element_type=jnp.float32)
    pf_run = jnp.minimum(run_id + float(RING - 1), nrun - 1.0)
    gg = (_row(rr) == pf_run).astype(jnp.float32)
    pf_e = jnp.dot(gg, er, preferred_element_type=jnp.float32)
    cslot = run_id - RING * jnp.floor(run_id * (1.0 / RING))
    pf_slot = pf_run - RING * jnp.floor(pf_run * (1.0 / RING))
    pf_do = (first * (run_id + float(RING - 1) <= nrun - 1.0)
             * (t_iota > 0))

    ti_c = lax.broadcasted_iota(jnp.int32, (NT, 8), 1)
    t_clampf = t_clamp.astype(jnp.float32)
    ti = jnp.where(ti_c == 0, tef,
         jnp.where(ti_c == 1, tv.astype(jnp.float32),
         jnp.where(ti_c == 2, first,
         jnp.where(ti_c == 3, cslot,
         jnp.where(ti_c == 4, pf_do,
         jnp.where(ti_c == 5, pf_e,
         jnp.where(ti_c == 6, pf_slot, t_clampf)))))))
    ti_ref[...] = ti.astype(jnp.int32)

    def _pick(r):
        idx = jnp.minimum(jnp.float32(r), nrun - 1.0)
        return jnp.sum(er * (rr == idx).astype(jnp.float32))

    hl = lax.broadcasted_iota(jnp.int32, (1, 8), 1)
    hdr = jnp.full((1, 8), 0.0, jnp.float32)
    for r in range(RING):
        hdr = jnp.where(hl == r, _pick(r), hdr)
    for r in range(1, RING):
        hdr = jnp.where(hl == RING + r - 1,
                        (nrun >= float(r + 1)).astype(jnp.float32), hdr)
    hdr_ref[...] = hdr.astype(jnp.int32)


def _router(x_flat, Wr):
    return pl.pallas_call(
        _router_body,
        out_shape=[
            jax.ShapeDtypeStruct((PAIRS, 1), jnp.int32),
            jax.ShapeDtypeStruct((S, 128), jnp.float32),
            jax.ShapeDtypeStruct((S, 128), jnp.float32),
            jax.ShapeDtypeStruct((NT, 8), jnp.int32),
            jax.ShapeDtypeStruct((1, 8), jnp.int32),
            jax.ShapeDtypeStruct((S, C // 2), jnp.int32),
        ],
    )(x_flat, Wr)


def _permute_body(x_hbm, posk_hbm, xp_hbm, pos_v, rows_v, sem):
    wid = lax.axis_index("s") * NC + lax.axis_index("c")
    per_w = PAIRS // NW
    ch = 64
    base = pl.multiple_of(wid * per_w, per_w)
    tok_base = pl.multiple_of((wid % NS) * per_w, per_w)
    for i in range(per_w // ch):
        pltpu.sync_copy(posk_hbm.at[pl.ds(base + i * ch, ch)], pos_v)
        pltpu.sync_copy(x_hbm.at[pl.ds(tok_base + i * ch, ch)], rows_v)
        pltpu.async_copy(rows_v, xp_hbm.at[pos_v], sem).wait()


def _sc_permute(x_flat, posk):
    mesh = plsc.VectorSubcoreMesh(core_axis_name="c", subcore_axis_name="s",
                                  num_cores=NC, num_subcores=NS)
    return pl.kernel(
        _permute_body,
        out_type=jax.ShapeDtypeStruct((NSLOT, C // 2), jnp.int32),
        mesh=mesh,
        scratch_types=[
            pltpu.VMEM((64,), jnp.int32),
            pltpu.VMEM((64, C // 2), jnp.int32),
            pltpu.SemaphoreType.DMA,
        ],
    )(x_flat, posk)


def _gemm_body(hdr_ref, ti_ref, x_ref, fc1_any, fc2_any, y_ref,
               w1buf, w2buf, s1, s2):
    t = pl.program_id(0)

    def issue(e, slot):
        pltpu.make_async_copy(fc1_any.at[e], w1buf.at[slot],
                              s1.at[slot]).start()
        pltpu.make_async_copy(fc2_any.at[e], w2buf.at[slot],
                              s2.at[slot]).start()

    def wait(slot):
        pltpu.make_async_copy(fc1_any.at[0], w1buf.at[slot],
                              s1.at[slot]).wait()
        pltpu.make_async_copy(fc2_any.at[0], w2buf.at[slot],
                              s2.at[slot]).wait()

    @pl.when(t == 0)
    def _():
        issue(hdr_ref[0], 0)
        for r in range(1, RING):
            @pl.when(hdr_ref[RING + r - 1] == 1)
            def _(r=r):
                issue(hdr_ref[r], r)

        wait(0)

    @pl.when((t > 0) & (ti_ref[t, 2] == 1))
    def _():
        @pl.when(ti_ref[t, 4] == 1)
        def _():
            issue(ti_ref[t, 5], ti_ref[t, 6])

        wait(ti_ref[t, 3])

    @pl.when(ti_ref[t, 1] == 1)
    def _():
        s = ti_ref[t, 3]
        xi = x_ref[...]
        xt = jnp.concatenate(
            [pltpu.unpack_elementwise(xi, index=0,
                                      packed_dtype=jnp.bfloat16,
                                      unpacked_dtype=jnp.float32),
             pltpu.unpack_elementwise(xi, index=1,
                                      packed_dtype=jnp.bfloat16,
                                      unpacked_dtype=jnp.float32)], axis=1)
        h = jnp.dot(xt, w1buf[s], preferred_element_type=jnp.float32)
        g, u = h[:, :FFN], h[:, FFN:]
        act = g * (1.0 / (1.0 + jnp.exp(-g))) * u
        yf = jnp.dot(act, w2buf[s], preferred_element_type=jnp.float32)
        y_ref[...] = pltpu.pack_elementwise(
            [yf[:, :C // 2], yf[:, C // 2:]], packed_dtype=jnp.bfloat16)


def _grouped_gemm(hdr, ti, x_perm, fc1_w, fc2_w):
    grid_spec = pltpu.PrefetchScalarGridSpec(
        num_scalar_prefetch=2,
        grid=(NT,),
        in_specs=[
            pl.BlockSpec((TM, C // 2), lambda t, hdr, ti: (ti[t, 7], 0)),
            pl.BlockSpec(memory_space=pl.ANY),
            pl.BlockSpec(memory_space=pl.ANY),
        ],
        out_specs=pl.BlockSpec((TM, C // 2),
                               lambda t, hdr, ti: (ti[t, 7], 0)),
        scratch_shapes=[
            pltpu.VMEM((RING, C, 2 * FFN), jnp.float32),
            pltpu.VMEM((RING, FFN, C), jnp.float32),
            pltpu.SemaphoreType.DMA((RING,)),
            pltpu.SemaphoreType.DMA((RING,)),
        ],
    )
    return pl.pallas_call(
        _gemm_body,
        grid_spec=grid_spec,
        out_shape=jax.ShapeDtypeStruct((NSLOT, C // 2), jnp.int32),
        compiler_params=pltpu.CompilerParams(
            dimension_semantics=("arbitrary",)),
    )(hdr, ti, x_perm, fc1_w, fc2_w)


def _unpermute_body(y_hbm, posk_hbm, y0_hbm, y1_hbm, pos_v, rows_v, sem):
    wid = lax.axis_index("s") * NC + lax.axis_index("c")
    per_t = S // NW
    base = pl.multiple_of(wid * per_t, per_t)
    for half, dst in ((0, y0_hbm), (1, y1_hbm)):
        pltpu.sync_copy(posk_hbm.at[pl.ds(half * S + base, per_t)], pos_v)
        pltpu.async_copy(y_hbm.at[pos_v], rows_v, sem).wait()
        pltpu.sync_copy(rows_v, dst.at[pl.ds(base, per_t)])


def _sc_unpermute(y, posk):
    mesh = plsc.VectorSubcoreMesh(core_axis_name="c", subcore_axis_name="s",
                                  num_cores=NC, num_subcores=NS)
    return pl.kernel(
        _unpermute_body,
        out_type=(jax.ShapeDtypeStruct((S, C // 2), jnp.int32),
                  jax.ShapeDtypeStruct((S, C // 2), jnp.int32)),
        mesh=mesh,
        scratch_types=[
            pltpu.VMEM((S // NW,), jnp.int32),
            pltpu.VMEM((S // NW, C // 2), jnp.int32),
            pltpu.SemaphoreType.DMA,
        ],
    )(y, posk)


def _shared_body(x_ref, wg_ref, wu_ref, wd_ref, o_ref):
    xt = x_ref[...]
    g = jnp.dot(xt, wg_ref[...], preferred_element_type=jnp.float32)
    u = jnp.dot(xt, wu_ref[...], preferred_element_type=jnp.float32)
    act = g * (1.0 / (1.0 + jnp.exp(-g))) * u
    o_ref[...] = jnp.dot(act, wd_ref[...], preferred_element_type=jnp.float32)


def _shared(x_flat, Wg_sh, Wu_sh, Wd_sh, half):
    tms = 256
    nt = S // tms // 2
    off = half * nt
    return pl.pallas_call(
        _shared_body,
        grid=(nt,),
        in_specs=[
            pl.BlockSpec((tms, C), lambda t: (t + off, 0)),
            pl.BlockSpec((C, FFN), lambda t: (0, 0)),
            pl.BlockSpec((C, FFN), lambda t: (0, 0)),
            pl.BlockSpec((FFN, C), lambda t: (0, 0)),
        ],
        out_specs=pl.BlockSpec((tms, C), lambda t: (t, 0)),
        out_shape=jax.ShapeDtypeStruct((S // 2, C), jnp.float32),
        compiler_params=pltpu.CompilerParams(
            dimension_semantics=("arbitrary",)),
    )(x_flat, Wg_sh, Wu_sh, Wd_sh)


def _unpack_bf16(yi):
    lo = pltpu.unpack_elementwise(yi, index=0, packed_dtype=jnp.bfloat16,
                                  unpacked_dtype=jnp.float32)
    hi = pltpu.unpack_elementwise(yi, index=1, packed_dtype=jnp.bfloat16,
                                  unpacked_dtype=jnp.float32)
    return jnp.concatenate([lo, hi], axis=1)


def _final_body(sh0_ref, sh1_ref, y0_ref, y1_ref, w0_ref, w1_ref, o_ref):
    t = pl.program_id(0)
    sh = jnp.where(t < 4, sh0_ref[...], sh1_ref[...])
    o_ref[...] = (sh + w0_ref[:, 0:1] * _unpack_bf16(y0_ref[...])
                  + w1_ref[:, 0:1] * _unpack_bf16(y1_ref[...]))


def _final(sh0, sh1, y0g, y1g, w0b, w1b):
    tms = 256
    return pl.pallas_call(
        _final_body,
        grid=(S // tms,),
        in_specs=[
            pl.BlockSpec((tms, C), lambda t: (jnp.minimum(t, 3), 0)),
            pl.BlockSpec((tms, C), lambda t: (jnp.maximum(t - 4, 0), 0)),
            pl.BlockSpec((tms, C // 2), lambda t: (t, 0)),
            pl.BlockSpec((tms, C // 2), lambda t: (t, 0)),
            pl.BlockSpec((tms, 128), lambda t: (t, 0)),
            pl.BlockSpec((tms, 128), lambda t: (t, 0)),
        ],
        out_specs=pl.BlockSpec((tms, C), lambda t: (t, 0)),
        out_shape=jax.ShapeDtypeStruct((S, C), jnp.float32),
        compiler_params=pltpu.CompilerParams(
            dimension_semantics=("arbitrary",)),
    )(sh0, sh1, y0g, y1g, w0b, w1b)


def kernel(x, Wr, Wg_sh, Wu_sh, Wd_sh, fc1_w, fc2_w):
    x_flat = x.reshape(S, C)
    posk2, w0b, w1b, ti, hdr2, xp16 = _router(x_flat, Wr)
    posk = posk2.reshape(PAIRS)
    x_perm = _sc_permute(xp16, posk)
    sh0 = _shared(x_flat, Wg_sh, Wu_sh, Wd_sh, 0)
    dep = (sh0[0, 0] * 0.0).astype(jnp.int32)
    y = _grouped_gemm(hdr2.reshape(8) + dep, ti, x_perm, fc1_w, fc2_w)
    sh1 = _shared(x_flat, Wg_sh, Wu_sh, Wd_sh, 1)
    y0g, y1g = _sc_unpermute(y, posk)
    out = _final(sh0, sh1, y0g, y1g, w0b, w1b)
    return out.reshape(B, T, C)
